# baseline (speedup 1.0000x reference)
import functools

import jax
import jax.numpy as jnp
from jax import lax
from jax.experimental import pallas as pl
from jax.experimental.pallas import tpu as pltpu
from jax.experimental.pallas import tpu_sc as plsc

B = 16384
F = 26
L = 50
V = 100000
D = 32

NC = 2
NS = 16
NW = NC * NS
SPW = B // NW
S = 16
NCHUNK = SPW // S
ROWS = S * (F + 1)
SEQN = S * L
GSUB = 128

VP = 100352
WU = 512
VBN = 196
NUNIT = (F + 1) * VBN
UPT = 166

_mesh = plsc.VectorSubcoreMesh(core_axis_name="c", subcore_axis_name="s")

_k1_buf = [
    pltpu.VMEM((D, WU), jnp.float32),
    pltpu.VMEM((D, 32), jnp.float32),
    pltpu.VMEM((WU * D,), jnp.float32),
    pltpu.SemaphoreType.DMA,
    pltpu.SemaphoreType.DMA,
]


@functools.partial(
    pl.kernel,
    out_type=(jax.ShapeDtypeStruct((F * VP * D,), jnp.float32),
              jax.ShapeDtypeStruct((VP * D,), jnp.float32)),
    mesh=_mesh,
    compiler_params=pltpu.CompilerParams(
        needs_layout_passes=False, use_tc_tiling_on_sc=True),
    scratch_types=_k1_buf + _k1_buf,
)
def _sc_detile(t832, seqT, out_tab, out_seq, *scr):
    bufs = (scr[:5], scr[5:])
    wid = lax.axis_index("s") * NC + lax.axis_index("c")
    lanes32 = lax.iota(jnp.int32, 16) * D

    def decode(k):
        u = wid + k * NW
        return u, u // VBN, u % VBN

    def in_copy(f, vb, in_v, tail_v, isem, wait):
        def go(src, r0):
            @pl.when(vb < VBN - 1)
            def _():
                cp = pltpu.make_async_copy(
                    src.at[pl.ds(r0, D), pl.ds(vb * WU, WU)], in_v, isem)
                cp.wait() if wait else cp.start()

            @pl.when(vb == VBN - 1)
            def _():
                t0 = (VBN - 1) * WU
                cps = [
                    pltpu.make_async_copy(
                        src.at[pl.ds(r0, D), pl.ds(t0, 128)],
                        in_v.at[pl.ds(0, D), pl.ds(0, 128)], isem),
                    pltpu.make_async_copy(
                        src.at[pl.ds(r0, D), pl.ds(t0 + 128, V - t0 - 128)],
                        tail_v, isem),
                ]
                for cp in cps:
                    cp.wait() if wait else cp.start()

        @pl.when(f < F)
        def _():
            go(t832, f * D)

        @pl.when(f == F)
        def _():
            go(seqT, 0)

    def out_copy(f, vb, out_v, osem, wait):
        @pl.when(f < F)
        def _():
            cp = pltpu.make_async_copy(
                out_v, out_tab.at[pl.ds((f * VP + vb * WU) * D, WU * D)], osem)
            cp.wait() if wait else cp.start()

        @pl.when(f == F)
        def _():
            cp = pltpu.make_async_copy(
                out_v, out_seq.at[pl.ds(vb * WU * D, WU * D)], osem)
            cp.wait() if wait else cp.start()

    def transpose(vb, in_v, tail_v, out_v):
        def seg_body(seg, carry):
            base = lanes32 + seg * (16 * D)
            for d in range(D):
                plsc.store_scatter(out_v, [base + d], in_v[d, pl.ds(seg * 16, 16)])
            return carry

        @pl.when(vb < VBN - 1)
        def _():
            lax.fori_loop(0, WU // 16, seg_body, 0)

        @pl.when(vb == VBN - 1)
        def _():
            lax.fori_loop(0, 8, seg_body, 0)
            for seg in (8, 9):
                base = lanes32 + seg * (16 * D)
                for d in range(D):
                    plsc.store_scatter(out_v, [base + d],
                                       tail_v[d, pl.ds((seg - 8) * 16, 16)])

    def stage1(k, bp):
        in_v, tail_v, out_v, isem, osem = bp
        u, f, vb = decode(k)

        @pl.when(u < NUNIT)
        def _():
            in_copy(f, vb, in_v, tail_v, isem, wait=False)

    def stage2(k, bp):
        in_v, tail_v, out_v, isem, osem = bp
        u, f, vb = decode(k)

        @pl.when(u < NUNIT)
        def _():
            in_copy(f, vb, in_v, tail_v, isem, wait=True)

            @pl.when(k >= 2)
            def _():
                _, f2, vb2 = decode(k - 2)
                out_copy(f2, vb2, out_v, osem, wait=True)

            transpose(vb, in_v, tail_v, out_v)
            out_copy(f, vb, out_v, osem, wait=False)

    stage1(0, bufs[0])

    def body(j, carry):
        k0 = 2 * j
        stage1(k0 + 1, bufs[1])
        stage2(k0, bufs[0])
        stage1(k0 + 2, bufs[0])
        stage2(k0 + 1, bufs[1])
        return carry

    lax.fori_loop(0, UPT // 2, body, 0)

    for kk in (UPT - 2, UPT - 1):
        u, f, vb = decode(kk)
        bp = bufs[kk % 2]

        @pl.when(u < NUNIT)
        def _():
            out_copy(f, vb, bp[2], bp[4], wait=True)


_k2_buf = [
    pltpu.VMEM((F, S), jnp.int32),
    pltpu.VMEM((ROWS,), jnp.int32),
    pltpu.VMEM((SEQN,), jnp.int32),
    pltpu.VMEM((ROWS, D), jnp.float32),
    pltpu.VMEM((SEQN, D), jnp.float32),
    pltpu.SemaphoreType.DMA,
    pltpu.SemaphoreType.DMA,
]


@functools.partial(
    pl.kernel,
    out_type=jax.ShapeDtypeStruct((B * (F + 1), D), jnp.float32),
    mesh=_mesh,
    compiler_params=pltpu.CompilerParams(
        needs_layout_passes=False, use_tc_tiling_on_sc=False),
    scratch_types=_k2_buf + _k2_buf + [
        pltpu.VMEM((1, D), jnp.float32),
    ],
)
def _sc_embed(spT, sq_ids, tables, seq_tab, out, *scr):
    b0, b1 = scr[:7], scr[7:14]
    row0_v = scr[14]
    wid = lax.axis_index("s") * NC + lax.axis_index("c")
    pltpu.sync_copy(seq_tab.at[pl.ds(0, 1)], row0_v)
    lanes = lax.iota(jnp.int32, 16)
    lanes27 = lanes * (F + 1)

    def stage1(c, bufs):
        ids2_v, idx_v, sqid_v, rows_v, seq_v, gsem, osem = bufs
        s_base = wid * SPW + c * S
        pltpu.sync_copy(spT.at[pl.ds(0, F), pl.ds(s_base, S)], ids2_v)
        pltpu.sync_copy(sq_ids.at[pl.ds(s_base * L, SEQN)], sqid_v)

        for f in range(F):
            idv = ids2_v[f, pl.ds(0, 16)]
            plsc.store_scatter(idx_v, [lanes27 + f], idv + f * VP)
        plsc.store_scatter(idx_v, [lanes27 + F], jnp.zeros((16,), jnp.int32))

        @pl.when(c >= 2)
        def _():
            old = (wid * SPW + (c - 2) * S) * (F + 1)
            pltpu.make_async_copy(rows_v, out.at[pl.ds(old, ROWS)], osem).wait()

        for off in range(0, ROWS, GSUB):
            n = min(GSUB, ROWS - off)
            pltpu.async_copy(
                tables.at[idx_v.at[pl.ds(off, n)]], rows_v.at[pl.ds(off, n)], gsem)
        for off in range(0, SEQN, GSUB):
            n = min(GSUB, SEQN - off)
            pltpu.async_copy(
                seq_tab.at[sqid_v.at[pl.ds(off, n)]], seq_v.at[pl.ds(off, n)], gsem)

    def stage2(c, bufs):
        ids2_v, idx_v, sqid_v, rows_v, seq_v, gsem, osem = bufs
        s_base = wid * SPW + c * S
        for off in range(0, ROWS, GSUB):
            n = min(GSUB, ROWS - off)
            pltpu.make_async_copy(
                tables.at[idx_v.at[pl.ds(off, n)]], rows_v.at[pl.ds(off, n)], gsem).wait()
        for off in range(0, SEQN, GSUB):
            n = min(GSUB, SEQN - off)
            pltpu.make_async_copy(
                seq_tab.at[sqid_v.at[pl.ds(off, n)]], seq_v.at[pl.ds(off, n)], gsem).wait()

        r0a = row0_v[0, pl.ds(0, 16)]
        r0b = row0_v[0, pl.ds(16, 16)]

        def sample_body(s, carry2):
            ob = s * L
            i0 = sqid_v[pl.ds(ob, 16)]
            i1 = sqid_v[pl.ds(ob + 16, 16)]
            i2 = sqid_v[pl.ds(ob + 32, 16)]
            i3 = sqid_v[pl.ds(ob + 34, 16)]
            cnt = (plsc.all_reduce_population_count(i0 != 0)
                   + plsc.all_reduce_population_count(i1 != 0)
                   + plsc.all_reduce_population_count(i2 != 0)
                   + plsc.all_reduce_population_count(
                       jnp.logical_and(i3 != 0, lanes >= 14)))
            acc0 = jnp.zeros((16,), jnp.float32)
            acc1 = jnp.zeros((16,), jnp.float32)
            for l in range(L):
                acc0 = acc0 + seq_v[ob + l, pl.ds(0, 16)]
                acc1 = acc1 + seq_v[ob + l, pl.ds(16, 16)]
            cf = cnt.astype(jnp.float32)
            nz = 50.0 - cf
            denom = jnp.maximum(cf, 1.0)
            p0 = jnp.where(cnt > 0, (acc0 - nz * r0a) / denom, 0.0)
            p1 = jnp.where(cnt > 0, (acc1 - nz * r0b) / denom, 0.0)
            orow = s * (F + 1) + F
            rows_v[orow, pl.ds(0, 16)] = p0
            rows_v[orow, pl.ds(16, 16)] = p1
            return carry2

        lax.fori_loop(0, S, sample_body, 0)
        pltpu.async_copy(rows_v, out.at[pl.ds(s_base * (F + 1), ROWS)], osem)

    stage1(0, b0)

    def body(k, carry):
        c0 = 2 * k
        stage1(c0 + 1, b1)
        stage2(c0, b0)

        @pl.when(c0 + 2 < NCHUNK)
        def _():
            stage1(c0 + 2, b0)

        stage2(c0 + 1, b1)
        return carry

    lax.fori_loop(0, NCHUNK // 2, body, 0)

    last0 = (wid * SPW + (NCHUNK - 2) * S) * (F + 1)
    last1 = (wid * SPW + (NCHUNK - 1) * S) * (F + 1)
    pltpu.make_async_copy(b0[3], out.at[pl.ds(last0, ROWS)], b0[6]).wait()
    pltpu.make_async_copy(b1[3], out.at[pl.ds(last1, ROWS)], b1[6]).wait()


def kernel(sparse_ids, seq_ids, sparse_tables, seq_table):
    t832 = jnp.swapaxes(sparse_tables, 1, 2).reshape(F * D, V)
    seqT = jnp.swapaxes(seq_table, 0, 1)
    ftab, fseq = _sc_detile(t832, seqT)
    spT = jnp.swapaxes(sparse_ids.astype(jnp.int32), 0, 1)
    sq = seq_ids.astype(jnp.int32).reshape(B * L)
    out = _sc_embed(spT, sq, ftab.reshape(F * VP, D), fseq.reshape(VP, D))
    return out.reshape(B, F + 1, D)


# speedup vs baseline: 1.8795x; 1.8795x over previous
import functools

import jax
import jax.numpy as jnp
from jax import lax
from jax.experimental import pallas as pl
from jax.experimental.pallas import tpu as pltpu
from jax.experimental.pallas import tpu_sc as plsc

B = 16384
F = 26
L = 50
V = 100000
D = 32

NC = 2
NS = 16
NW = NC * NS
SPW = B // NW
S = 16
NCHUNK = SPW // S
ROWS = S * (F + 1)
SEQN = S * L
GSUB = 128

VP = 100352
WU = 512
VBN = 196
NUNIT = (F + 1) * VBN
UPT = 166

_mesh = plsc.VectorSubcoreMesh(core_axis_name="c", subcore_axis_name="s")

_k1_buf = [
    pltpu.VMEM((D, WU), jnp.float32),
    pltpu.VMEM((D, 32), jnp.float32),
    pltpu.VMEM((WU * D,), jnp.float32),
    pltpu.SemaphoreType.DMA,
    pltpu.SemaphoreType.DMA,
]


@functools.partial(
    pl.kernel,
    out_type=(jax.ShapeDtypeStruct((F * VP * D,), jnp.float32),
              jax.ShapeDtypeStruct((VP * D,), jnp.float32)),
    mesh=_mesh,
    compiler_params=pltpu.CompilerParams(
        needs_layout_passes=False, use_tc_tiling_on_sc=True),
    scratch_types=_k1_buf + _k1_buf,
)
def _sc_detile(t832, seqT, out_tab, out_seq, *scr):
    bufs = (scr[:5], scr[5:])
    wid = lax.axis_index("s") * NC + lax.axis_index("c")
    lanes32 = lax.iota(jnp.int32, 16) * D

    def decode(k):
        u = wid + k * NW
        return u, u // VBN, u % VBN

    def in_copy(f, vb, in_v, tail_v, isem, wait):
        def go(src, r0):
            @pl.when(vb < VBN - 1)
            def _():
                cp = pltpu.make_async_copy(
                    src.at[pl.ds(r0, D), pl.ds(vb * WU, WU)], in_v, isem)
                cp.wait() if wait else cp.start()

            @pl.when(vb == VBN - 1)
            def _():
                t0 = (VBN - 1) * WU
                cps = [
                    pltpu.make_async_copy(
                        src.at[pl.ds(r0, D), pl.ds(t0, 128)],
                        in_v.at[pl.ds(0, D), pl.ds(0, 128)], isem),
                    pltpu.make_async_copy(
                        src.at[pl.ds(r0, D), pl.ds(t0 + 128, V - t0 - 128)],
                        tail_v, isem),
                ]
                for cp in cps:
                    cp.wait() if wait else cp.start()

        @pl.when(f < F)
        def _():
            go(t832, f * D)

        @pl.when(f == F)
        def _():
            go(seqT, 0)

    def out_copy(f, vb, out_v, osem, wait):
        @pl.when(f < F)
        def _():
            cp = pltpu.make_async_copy(
                out_v, out_tab.at[pl.ds((f * VP + vb * WU) * D, WU * D)], osem)
            cp.wait() if wait else cp.start()

        @pl.when(f == F)
        def _():
            cp = pltpu.make_async_copy(
                out_v, out_seq.at[pl.ds(vb * WU * D, WU * D)], osem)
            cp.wait() if wait else cp.start()

    def transpose(vb, in_v, tail_v, out_v):
        lanes_ = lax.iota(jnp.int32, 16)

        def diag(src, sl):
            sl32 = sl * D
            for k in range(D):
                dvec = jnp.bitwise_and(lanes_ + k, D - 1)
                val = plsc.load_gather(src, [dvec, sl])
                plsc.store_scatter(out_v, [sl32 + dvec], val)

        def seg_body(seg, carry):
            diag(in_v, lanes_ + seg * 16)
            return carry

        @pl.when(vb < VBN - 1)
        def _():
            lax.fori_loop(0, WU // 16, seg_body, 0)

        @pl.when(vb == VBN - 1)
        def _():
            lax.fori_loop(0, 8, seg_body, 0)
            for seg in (8, 9):
                sl = lanes_ + seg * 16
                sl32 = sl * D
                for k in range(D):
                    dvec = jnp.bitwise_and(lanes_ + k, D - 1)
                    val = plsc.load_gather(tail_v, [dvec, sl - 128])
                    plsc.store_scatter(out_v, [sl32 + dvec], val)

    def stage1(k, bp):
        in_v, tail_v, out_v, isem, osem = bp
        u, f, vb = decode(k)

        @pl.when(u < NUNIT)
        def _():
            in_copy(f, vb, in_v, tail_v, isem, wait=False)

    def stage2(k, bp):
        in_v, tail_v, out_v, isem, osem = bp
        u, f, vb = decode(k)

        @pl.when(u < NUNIT)
        def _():
            in_copy(f, vb, in_v, tail_v, isem, wait=True)

            @pl.when(k >= 2)
            def _():
                _, f2, vb2 = decode(k - 2)
                out_copy(f2, vb2, out_v, osem, wait=True)

            transpose(vb, in_v, tail_v, out_v)
            out_copy(f, vb, out_v, osem, wait=False)

    stage1(0, bufs[0])

    def body(j, carry):
        k0 = 2 * j
        stage1(k0 + 1, bufs[1])
        stage2(k0, bufs[0])
        stage1(k0 + 2, bufs[0])
        stage2(k0 + 1, bufs[1])
        return carry

    lax.fori_loop(0, UPT // 2, body, 0)

    for kk in (UPT - 2, UPT - 1):
        u, f, vb = decode(kk)
        bp = bufs[kk % 2]

        @pl.when(u < NUNIT)
        def _():
            out_copy(f, vb, bp[2], bp[4], wait=True)


_k2_buf = [
    pltpu.VMEM((F, S), jnp.int32),
    pltpu.VMEM((ROWS,), jnp.int32),
    pltpu.VMEM((SEQN,), jnp.int32),
    pltpu.VMEM((ROWS, D), jnp.float32),
    pltpu.VMEM((SEQN, D), jnp.float32),
    pltpu.SemaphoreType.DMA,
    pltpu.SemaphoreType.DMA,
]


@functools.partial(
    pl.kernel,
    out_type=jax.ShapeDtypeStruct((B * (F + 1), D), jnp.float32),
    mesh=_mesh,
    compiler_params=pltpu.CompilerParams(
        needs_layout_passes=False, use_tc_tiling_on_sc=False),
    scratch_types=_k2_buf + _k2_buf + [
        pltpu.VMEM((1, D), jnp.float32),
    ],
)
def _sc_embed(spT, sq_ids, tables, seq_tab, out, *scr):
    b0, b1 = scr[:7], scr[7:14]
    row0_v = scr[14]
    wid = lax.axis_index("s") * NC + lax.axis_index("c")
    pltpu.sync_copy(seq_tab.at[pl.ds(0, 1)], row0_v)
    lanes = lax.iota(jnp.int32, 16)
    lanes27 = lanes * (F + 1)

    def stage1(c, bufs):
        ids2_v, idx_v, sqid_v, rows_v, seq_v, gsem, osem = bufs
        s_base = wid * SPW + c * S
        pltpu.sync_copy(spT.at[pl.ds(0, F), pl.ds(s_base, S)], ids2_v)
        pltpu.sync_copy(sq_ids.at[pl.ds(s_base * L, SEQN)], sqid_v)

        for f in range(F):
            idv = ids2_v[f, pl.ds(0, 16)]
            plsc.store_scatter(idx_v, [lanes27 + f], idv + f * VP)
        plsc.store_scatter(idx_v, [lanes27 + F], jnp.zeros((16,), jnp.int32))

        @pl.when(c >= 2)
        def _():
            old = (wid * SPW + (c - 2) * S) * (F + 1)
            pltpu.make_async_copy(rows_v, out.at[pl.ds(old, ROWS)], osem).wait()

        for off in range(0, ROWS, GSUB):
            n = min(GSUB, ROWS - off)
            pltpu.async_copy(
                tables.at[idx_v.at[pl.ds(off, n)]], rows_v.at[pl.ds(off, n)], gsem)
        for off in range(0, SEQN, GSUB):
            n = min(GSUB, SEQN - off)
            pltpu.async_copy(
                seq_tab.at[sqid_v.at[pl.ds(off, n)]], seq_v.at[pl.ds(off, n)], gsem)

    def stage2(c, bufs):
        ids2_v, idx_v, sqid_v, rows_v, seq_v, gsem, osem = bufs
        s_base = wid * SPW + c * S
        for off in range(0, ROWS, GSUB):
            n = min(GSUB, ROWS - off)
            pltpu.make_async_copy(
                tables.at[idx_v.at[pl.ds(off, n)]], rows_v.at[pl.ds(off, n)], gsem).wait()
        for off in range(0, SEQN, GSUB):
            n = min(GSUB, SEQN - off)
            pltpu.make_async_copy(
                seq_tab.at[sqid_v.at[pl.ds(off, n)]], seq_v.at[pl.ds(off, n)], gsem).wait()

        r0a = row0_v[0, pl.ds(0, 16)]
        r0b = row0_v[0, pl.ds(16, 16)]

        def sample_body(s, carry2):
            ob = s * L
            i0 = sqid_v[pl.ds(ob, 16)]
            i1 = sqid_v[pl.ds(ob + 16, 16)]
            i2 = sqid_v[pl.ds(ob + 32, 16)]
            i3 = sqid_v[pl.ds(ob + 34, 16)]
            cnt = (plsc.all_reduce_population_count(i0 != 0)
                   + plsc.all_reduce_population_count(i1 != 0)
                   + plsc.all_reduce_population_count(i2 != 0)
                   + plsc.all_reduce_population_count(
                       jnp.logical_and(i3 != 0, lanes >= 14)))
            acc0 = jnp.zeros((16,), jnp.float32)
            acc1 = jnp.zeros((16,), jnp.float32)
            for l in range(L):
                acc0 = acc0 + seq_v[ob + l, pl.ds(0, 16)]
                acc1 = acc1 + seq_v[ob + l, pl.ds(16, 16)]
            cf = cnt.astype(jnp.float32)
            nz = 50.0 - cf
            denom = jnp.maximum(cf, 1.0)
            p0 = jnp.where(cnt > 0, (acc0 - nz * r0a) / denom, 0.0)
            p1 = jnp.where(cnt > 0, (acc1 - nz * r0b) / denom, 0.0)
            orow = s * (F + 1) + F
            rows_v[orow, pl.ds(0, 16)] = p0
            rows_v[orow, pl.ds(16, 16)] = p1
            return carry2

        lax.fori_loop(0, S, sample_body, 0)
        pltpu.async_copy(rows_v, out.at[pl.ds(s_base * (F + 1), ROWS)], osem)

    stage1(0, b0)

    def body(k, carry):
        c0 = 2 * k
        stage1(c0 + 1, b1)
        stage2(c0, b0)

        @pl.when(c0 + 2 < NCHUNK)
        def _():
            stage1(c0 + 2, b0)

        stage2(c0 + 1, b1)
        return carry

    lax.fori_loop(0, NCHUNK // 2, body, 0)

    last0 = (wid * SPW + (NCHUNK - 2) * S) * (F + 1)
    last1 = (wid * SPW + (NCHUNK - 1) * S) * (F + 1)
    pltpu.make_async_copy(b0[3], out.at[pl.ds(last0, ROWS)], b0[6]).wait()
    pltpu.make_async_copy(b1[3], out.at[pl.ds(last1, ROWS)], b1[6]).wait()


def kernel(sparse_ids, seq_ids, sparse_tables, seq_table):
    t832 = jnp.swapaxes(sparse_tables, 1, 2).reshape(F * D, V)
    seqT = jnp.swapaxes(seq_table, 0, 1)
    ftab, fseq = _sc_detile(t832, seqT)
    spT = jnp.swapaxes(sparse_ids.astype(jnp.int32), 0, 1)
    sq = seq_ids.astype(jnp.int32).reshape(B * L)
    out = _sc_embed(spT, sq, ftab.reshape(F * VP, D), fseq.reshape(VP, D))
    return out.reshape(B, F + 1, D)


# speedup vs baseline: 2.4221x; 1.2887x over previous
import functools

import jax
import jax.numpy as jnp
from jax import lax
from jax.experimental import pallas as pl
from jax.experimental.pallas import tpu as pltpu
from jax.experimental.pallas import tpu_sc as plsc

B = 16384
F = 26
L = 50
V = 100000
D = 32

NC = 2
NS = 16
NW = NC * NS
SPW = B // NW
S = 16
NCHUNK = SPW // S
ROWS = S * (F + 1)
SEQN = S * L
GSUB = 128

VP = 100352
WU = 512
VBN = 196
NUNIT = (F + 1) * VBN
UPT = 166

_mesh = plsc.VectorSubcoreMesh(core_axis_name="c", subcore_axis_name="s")

_k1_buf = [
    pltpu.VMEM((D, WU), jnp.float32),
    pltpu.VMEM((D, 32), jnp.float32),
    pltpu.VMEM((WU * D,), jnp.float32),
    pltpu.SemaphoreType.DMA,
    pltpu.SemaphoreType.DMA,
]


@functools.partial(
    pl.kernel,
    out_type=(jax.ShapeDtypeStruct((F * VP * D,), jnp.float32),
              jax.ShapeDtypeStruct((VP * D,), jnp.float32)),
    mesh=_mesh,
    compiler_params=pltpu.CompilerParams(
        needs_layout_passes=False, use_tc_tiling_on_sc=True),
    scratch_types=_k1_buf + _k1_buf,
)
def _sc_detile(t832, seqT, out_tab, out_seq, *scr):
    bufs = (scr[:5], scr[5:])
    wid = lax.axis_index("s") * NC + lax.axis_index("c")
    lanes32 = lax.iota(jnp.int32, 16) * D

    def decode(k):
        u = wid + k * NW
        return u, u // VBN, u % VBN

    def in_copy(f, vb, in_v, tail_v, isem, wait):
        def go(src, r0):
            @pl.when(vb < VBN - 1)
            def _():
                cp = pltpu.make_async_copy(
                    src.at[pl.ds(r0, D), pl.ds(vb * WU, WU)], in_v, isem)
                cp.wait() if wait else cp.start()

            @pl.when(vb == VBN - 1)
            def _():
                t0 = (VBN - 1) * WU
                cps = [
                    pltpu.make_async_copy(
                        src.at[pl.ds(r0, D), pl.ds(t0, 128)],
                        in_v.at[pl.ds(0, D), pl.ds(0, 128)], isem),
                    pltpu.make_async_copy(
                        src.at[pl.ds(r0, D), pl.ds(t0 + 128, V - t0 - 128)],
                        tail_v, isem),
                ]
                for cp in cps:
                    cp.wait() if wait else cp.start()

        @pl.when(f < F)
        def _():
            go(t832, f * D)

        @pl.when(f == F)
        def _():
            go(seqT, 0)

    def out_copy(f, vb, out_v, osem, wait):
        @pl.when(f < F)
        def _():
            cp = pltpu.make_async_copy(
                out_v, out_tab.at[pl.ds((f * VP + vb * WU) * D, WU * D)], osem)
            cp.wait() if wait else cp.start()

        @pl.when(f == F)
        def _():
            cp = pltpu.make_async_copy(
                out_v, out_seq.at[pl.ds(vb * WU * D, WU * D)], osem)
            cp.wait() if wait else cp.start()

    def transpose(vb, in_v, tail_v, out_v):
        lanes_ = lax.iota(jnp.int32, 16)

        def diag(src, sl):
            sl32 = sl * D
            for k in range(D):
                dvec = jnp.bitwise_and(lanes_ + k, D - 1)
                val = plsc.load_gather(src, [dvec, sl])
                plsc.store_scatter(out_v, [sl32 + dvec], val)

        @pl.when(vb < VBN - 1)
        def _():
            @plsc.parallel_loop(0, WU // 16, unroll=2)
            def _(seg):
                diag(in_v, lanes_ + seg * 16)

        @pl.when(vb == VBN - 1)
        def _():
            @plsc.parallel_loop(0, 8, unroll=2)
            def _(seg):
                diag(in_v, lanes_ + seg * 16)
            for seg in (8, 9):
                sl = lanes_ + seg * 16
                sl32 = sl * D
                for k in range(D):
                    dvec = jnp.bitwise_and(lanes_ + k, D - 1)
                    val = plsc.load_gather(tail_v, [dvec, sl - 128])
                    plsc.store_scatter(out_v, [sl32 + dvec], val)

    def stage1(k, bp):
        in_v, tail_v, out_v, isem, osem = bp
        u, f, vb = decode(k)

        @pl.when(u < NUNIT)
        def _():
            in_copy(f, vb, in_v, tail_v, isem, wait=False)

    def stage2(k, bp):
        in_v, tail_v, out_v, isem, osem = bp
        u, f, vb = decode(k)

        @pl.when(u < NUNIT)
        def _():
            in_copy(f, vb, in_v, tail_v, isem, wait=True)

            @pl.when(k >= 2)
            def _():
                _, f2, vb2 = decode(k - 2)
                out_copy(f2, vb2, out_v, osem, wait=True)

            transpose(vb, in_v, tail_v, out_v)
            out_copy(f, vb, out_v, osem, wait=False)

    stage1(0, bufs[0])

    def body(j, carry):
        k0 = 2 * j
        stage1(k0 + 1, bufs[1])
        stage2(k0, bufs[0])
        stage1(k0 + 2, bufs[0])
        stage2(k0 + 1, bufs[1])
        return carry

    lax.fori_loop(0, UPT // 2, body, 0)

    for kk in (UPT - 2, UPT - 1):
        u, f, vb = decode(kk)
        bp = bufs[kk % 2]

        @pl.when(u < NUNIT)
        def _():
            out_copy(f, vb, bp[2], bp[4], wait=True)


_k2_buf = [
    pltpu.VMEM((F, S), jnp.int32),
    pltpu.VMEM((ROWS,), jnp.int32),
    pltpu.VMEM((SEQN,), jnp.int32),
    pltpu.VMEM((ROWS, D), jnp.float32),
    pltpu.VMEM((SEQN, D), jnp.float32),
    pltpu.SemaphoreType.DMA,
    pltpu.SemaphoreType.DMA,
]


@functools.partial(
    pl.kernel,
    out_type=jax.ShapeDtypeStruct((B * (F + 1), D), jnp.float32),
    mesh=_mesh,
    compiler_params=pltpu.CompilerParams(
        needs_layout_passes=False, use_tc_tiling_on_sc=False),
    scratch_types=_k2_buf + _k2_buf + [
        pltpu.VMEM((1, D), jnp.float32),
    ],
)
def _sc_embed(spT, sq_ids, tables, seq_tab, out, *scr):
    b0, b1 = scr[:7], scr[7:14]
    row0_v = scr[14]
    wid = lax.axis_index("s") * NC + lax.axis_index("c")
    pltpu.sync_copy(seq_tab.at[pl.ds(0, 1)], row0_v)
    lanes = lax.iota(jnp.int32, 16)
    lanes27 = lanes * (F + 1)

    def stage1(c, bufs):
        ids2_v, idx_v, sqid_v, rows_v, seq_v, gsem, osem = bufs
        s_base = wid * SPW + c * S
        pltpu.sync_copy(spT.at[pl.ds(0, F), pl.ds(s_base, S)], ids2_v)
        pltpu.sync_copy(sq_ids.at[pl.ds(s_base * L, SEQN)], sqid_v)

        for f in range(F):
            idv = ids2_v[f, pl.ds(0, 16)]
            plsc.store_scatter(idx_v, [lanes27 + f], idv + f * VP)
        plsc.store_scatter(idx_v, [lanes27 + F], jnp.zeros((16,), jnp.int32))

        @pl.when(c >= 2)
        def _():
            old = (wid * SPW + (c - 2) * S) * (F + 1)
            pltpu.make_async_copy(rows_v, out.at[pl.ds(old, ROWS)], osem).wait()

        for off in range(0, ROWS, GSUB):
            n = min(GSUB, ROWS - off)
            pltpu.async_copy(
                tables.at[idx_v.at[pl.ds(off, n)]], rows_v.at[pl.ds(off, n)], gsem)
        for off in range(0, SEQN, GSUB):
            n = min(GSUB, SEQN - off)
            pltpu.async_copy(
                seq_tab.at[sqid_v.at[pl.ds(off, n)]], seq_v.at[pl.ds(off, n)], gsem)

    def stage2(c, bufs):
        ids2_v, idx_v, sqid_v, rows_v, seq_v, gsem, osem = bufs
        s_base = wid * SPW + c * S
        for off in range(0, ROWS, GSUB):
            n = min(GSUB, ROWS - off)
            pltpu.make_async_copy(
                tables.at[idx_v.at[pl.ds(off, n)]], rows_v.at[pl.ds(off, n)], gsem).wait()
        for off in range(0, SEQN, GSUB):
            n = min(GSUB, SEQN - off)
            pltpu.make_async_copy(
                seq_tab.at[sqid_v.at[pl.ds(off, n)]], seq_v.at[pl.ds(off, n)], gsem).wait()

        r0a = row0_v[0, pl.ds(0, 16)]
        r0b = row0_v[0, pl.ds(16, 16)]

        @plsc.parallel_loop(0, S)
        def sample_body(s):
            ob = s * L
            i0 = sqid_v[pl.ds(ob, 16)]
            i1 = sqid_v[pl.ds(ob + 16, 16)]
            i2 = sqid_v[pl.ds(ob + 32, 16)]
            i3 = sqid_v[pl.ds(ob + 34, 16)]
            cnt = (plsc.all_reduce_population_count(i0 != 0)
                   + plsc.all_reduce_population_count(i1 != 0)
                   + plsc.all_reduce_population_count(i2 != 0)
                   + plsc.all_reduce_population_count(
                       jnp.logical_and(i3 != 0, lanes >= 14)))
            acc0 = jnp.zeros((16,), jnp.float32)
            acc1 = jnp.zeros((16,), jnp.float32)
            for l in range(L):
                acc0 = acc0 + seq_v[ob + l, pl.ds(0, 16)]
                acc1 = acc1 + seq_v[ob + l, pl.ds(16, 16)]
            cf = cnt.astype(jnp.float32)
            nz = 50.0 - cf
            denom = jnp.maximum(cf, 1.0)
            p0 = jnp.where(cnt > 0, (acc0 - nz * r0a) / denom, 0.0)
            p1 = jnp.where(cnt > 0, (acc1 - nz * r0b) / denom, 0.0)
            orow = s * (F + 1) + F
            rows_v[orow, pl.ds(0, 16)] = p0
            rows_v[orow, pl.ds(16, 16)] = p1

        pltpu.async_copy(rows_v, out.at[pl.ds(s_base * (F + 1), ROWS)], osem)

    stage1(0, b0)

    def body(k, carry):
        c0 = 2 * k
        stage1(c0 + 1, b1)
        stage2(c0, b0)

        @pl.when(c0 + 2 < NCHUNK)
        def _():
            stage1(c0 + 2, b0)

        stage2(c0 + 1, b1)
        return carry

    lax.fori_loop(0, NCHUNK // 2, body, 0)

    last0 = (wid * SPW + (NCHUNK - 2) * S) * (F + 1)
    last1 = (wid * SPW + (NCHUNK - 1) * S) * (F + 1)
    pltpu.make_async_copy(b0[3], out.at[pl.ds(last0, ROWS)], b0[6]).wait()
    pltpu.make_async_copy(b1[3], out.at[pl.ds(last1, ROWS)], b1[6]).wait()


def kernel(sparse_ids, seq_ids, sparse_tables, seq_table):
    t832 = jnp.swapaxes(sparse_tables, 1, 2).reshape(F * D, V)
    seqT = jnp.swapaxes(seq_table, 0, 1)
    ftab, fseq = _sc_detile(t832, seqT)
    spT = jnp.swapaxes(sparse_ids.astype(jnp.int32), 0, 1)
    sq = seq_ids.astype(jnp.int32).reshape(B * L)
    out = _sc_embed(spT, sq, ftab.reshape(F * VP, D), fseq.reshape(VP, D))
    return out.reshape(B, F + 1, D)


# speedup vs baseline: 2.6946x; 1.1125x over previous
import functools

import jax
import jax.numpy as jnp
from jax import lax
from jax.experimental import pallas as pl
from jax.experimental.pallas import tpu as pltpu
from jax.experimental.pallas import tpu_sc as plsc

B = 16384
F = 26
L = 50
V = 100000
D = 32

NC = 2
NS = 16
NW = NC * NS
SPW = B // NW
S = 16
NCHUNK = SPW // S
ROWS = S * (F + 1)
SEQN = S * L
GSUB = 128

VP = 100352
WU = 512
VBN = 196
NUNIT = (F + 1) * VBN
UPT = 166

_mesh = plsc.VectorSubcoreMesh(core_axis_name="c", subcore_axis_name="s")

_k1_buf = [
    pltpu.VMEM((D, WU), jnp.float32),
    pltpu.VMEM((D, 32), jnp.float32),
    pltpu.VMEM((WU * D,), jnp.float32),
    pltpu.SemaphoreType.DMA,
    pltpu.SemaphoreType.DMA,
]


@functools.partial(
    pl.kernel,
    out_type=(jax.ShapeDtypeStruct((F * VP * D,), jnp.float32),
              jax.ShapeDtypeStruct((VP * D,), jnp.float32)),
    mesh=_mesh,
    compiler_params=pltpu.CompilerParams(
        needs_layout_passes=False, use_tc_tiling_on_sc=True),
    scratch_types=_k1_buf + _k1_buf,
)
def _sc_detile(t832, seqT, out_tab, out_seq, *scr):
    bufs = (scr[:5], scr[5:])
    wid = lax.axis_index("s") * NC + lax.axis_index("c")
    lanes32 = lax.iota(jnp.int32, 16) * D

    def decode(k):
        u = wid + k * NW
        return u, u // VBN, u % VBN

    def in_copy(f, vb, in_v, tail_v, isem, wait):
        def go(src, r0):
            @pl.when(vb < VBN - 1)
            def _():
                cp = pltpu.make_async_copy(
                    src.at[pl.ds(r0, D), pl.ds(vb * WU, WU)], in_v, isem)
                cp.wait() if wait else cp.start()

            @pl.when(vb == VBN - 1)
            def _():
                t0 = (VBN - 1) * WU
                cps = [
                    pltpu.make_async_copy(
                        src.at[pl.ds(r0, D), pl.ds(t0, 128)],
                        in_v.at[pl.ds(0, D), pl.ds(0, 128)], isem),
                    pltpu.make_async_copy(
                        src.at[pl.ds(r0, D), pl.ds(t0 + 128, V - t0 - 128)],
                        tail_v, isem),
                ]
                for cp in cps:
                    cp.wait() if wait else cp.start()

        @pl.when(f < F)
        def _():
            go(t832, f * D)

        @pl.when(f == F)
        def _():
            go(seqT, 0)

    def out_copy(f, vb, out_v, osem, wait):
        @pl.when(f < F)
        def _():
            cp = pltpu.make_async_copy(
                out_v, out_tab.at[pl.ds((f * VP + vb * WU) * D, WU * D)], osem)
            cp.wait() if wait else cp.start()

        @pl.when(f == F)
        def _():
            cp = pltpu.make_async_copy(
                out_v, out_seq.at[pl.ds(vb * WU * D, WU * D)], osem)
            cp.wait() if wait else cp.start()

    def transpose(vb, in_v, tail_v, out_v):
        lanes_ = lax.iota(jnp.int32, 16)

        def diag(src, sl):
            sl32 = sl * D
            for k in range(D):
                dvec = jnp.bitwise_and(lanes_ + k, D - 1)
                val = plsc.load_gather(src, [dvec, sl])
                plsc.store_scatter(out_v, [sl32 + dvec], val)

        @pl.when(vb < VBN - 1)
        def _():
            @plsc.parallel_loop(0, WU // 16, unroll=4)
            def _(seg):
                diag(in_v, lanes_ + seg * 16)

        @pl.when(vb == VBN - 1)
        def _():
            @plsc.parallel_loop(0, 8, unroll=2)
            def _(seg):
                diag(in_v, lanes_ + seg * 16)
            for seg in (8, 9):
                sl = lanes_ + seg * 16
                sl32 = sl * D
                for k in range(D):
                    dvec = jnp.bitwise_and(lanes_ + k, D - 1)
                    val = plsc.load_gather(tail_v, [dvec, sl - 128])
                    plsc.store_scatter(out_v, [sl32 + dvec], val)

    def stage1(k, bp):
        in_v, tail_v, out_v, isem, osem = bp
        u, f, vb = decode(k)

        @pl.when(u < NUNIT)
        def _():
            in_copy(f, vb, in_v, tail_v, isem, wait=False)

    def stage2(k, bp):
        in_v, tail_v, out_v, isem, osem = bp
        u, f, vb = decode(k)

        @pl.when(u < NUNIT)
        def _():
            in_copy(f, vb, in_v, tail_v, isem, wait=True)

            @pl.when(k >= 2)
            def _():
                _, f2, vb2 = decode(k - 2)
                out_copy(f2, vb2, out_v, osem, wait=True)

            transpose(vb, in_v, tail_v, out_v)
            out_copy(f, vb, out_v, osem, wait=False)

    stage1(0, bufs[0])

    def body(j, carry):
        k0 = 2 * j
        stage1(k0 + 1, bufs[1])
        stage2(k0, bufs[0])
        stage1(k0 + 2, bufs[0])
        stage2(k0 + 1, bufs[1])
        return carry

    lax.fori_loop(0, UPT // 2, body, 0)

    for kk in (UPT - 2, UPT - 1):
        u, f, vb = decode(kk)
        bp = bufs[kk % 2]

        @pl.when(u < NUNIT)
        def _():
            out_copy(f, vb, bp[2], bp[4], wait=True)


_k2_buf = [
    pltpu.VMEM((F, S), jnp.int32),
    pltpu.VMEM((ROWS,), jnp.int32),
    pltpu.VMEM((SEQN,), jnp.int32),
    pltpu.VMEM((ROWS, D), jnp.float32),
    pltpu.VMEM((SEQN, D), jnp.float32),
    pltpu.SemaphoreType.DMA,
    pltpu.SemaphoreType.DMA,
]


@functools.partial(
    pl.kernel,
    out_type=jax.ShapeDtypeStruct((B * (F + 1), D), jnp.float32),
    mesh=_mesh,
    compiler_params=pltpu.CompilerParams(
        needs_layout_passes=False, use_tc_tiling_on_sc=False),
    scratch_types=_k2_buf + _k2_buf + [
        pltpu.VMEM((1, D), jnp.float32),
    ],
)
def _sc_embed(spT, sq_ids, tables, seq_tab, out, *scr):
    b0, b1 = scr[:7], scr[7:14]
    row0_v = scr[14]
    wid = lax.axis_index("s") * NC + lax.axis_index("c")
    pltpu.sync_copy(seq_tab.at[pl.ds(0, 1)], row0_v)
    lanes = lax.iota(jnp.int32, 16)
    lanes27 = lanes * (F + 1)

    def stage1(c, bufs):
        ids2_v, idx_v, sqid_v, rows_v, seq_v, gsem, osem = bufs
        s_base = wid * SPW + c * S
        pltpu.sync_copy(spT.at[pl.ds(0, F), pl.ds(s_base, S)], ids2_v)
        pltpu.sync_copy(sq_ids.at[pl.ds(s_base * L, SEQN)], sqid_v)

        for f in range(F):
            idv = ids2_v[f, pl.ds(0, 16)]
            plsc.store_scatter(idx_v, [lanes27 + f], idv + f * VP)
        plsc.store_scatter(idx_v, [lanes27 + F], jnp.zeros((16,), jnp.int32))

        @pl.when(c >= 2)
        def _():
            old = (wid * SPW + (c - 2) * S) * (F + 1)
            pltpu.make_async_copy(rows_v, out.at[pl.ds(old, ROWS)], osem).wait()

        for off in range(0, ROWS, GSUB):
            n = min(GSUB, ROWS - off)
            pltpu.async_copy(
                tables.at[idx_v.at[pl.ds(off, n)]], rows_v.at[pl.ds(off, n)], gsem)
        for off in range(0, SEQN, GSUB):
            n = min(GSUB, SEQN - off)
            pltpu.async_copy(
                seq_tab.at[sqid_v.at[pl.ds(off, n)]], seq_v.at[pl.ds(off, n)], gsem)

    def stage2(c, bufs):
        ids2_v, idx_v, sqid_v, rows_v, seq_v, gsem, osem = bufs
        s_base = wid * SPW + c * S
        for off in range(0, ROWS, GSUB):
            n = min(GSUB, ROWS - off)
            pltpu.make_async_copy(
                tables.at[idx_v.at[pl.ds(off, n)]], rows_v.at[pl.ds(off, n)], gsem).wait()
        for off in range(0, SEQN, GSUB):
            n = min(GSUB, SEQN - off)
            pltpu.make_async_copy(
                seq_tab.at[sqid_v.at[pl.ds(off, n)]], seq_v.at[pl.ds(off, n)], gsem).wait()

        r0a = row0_v[0, pl.ds(0, 16)]
        r0b = row0_v[0, pl.ds(16, 16)]

        @plsc.parallel_loop(0, S)
        def sample_body(s):
            ob = s * L
            i0 = sqid_v[pl.ds(ob, 16)]
            i1 = sqid_v[pl.ds(ob + 16, 16)]
            i2 = sqid_v[pl.ds(ob + 32, 16)]
            i3 = sqid_v[pl.ds(ob + 34, 16)]
            cnt = (plsc.all_reduce_population_count(i0 != 0)
                   + plsc.all_reduce_population_count(i1 != 0)
                   + plsc.all_reduce_population_count(i2 != 0)
                   + plsc.all_reduce_population_count(
                       jnp.logical_and(i3 != 0, lanes >= 14)))
            acc0 = jnp.zeros((16,), jnp.float32)
            acc1 = jnp.zeros((16,), jnp.float32)
            for l in range(L):
                acc0 = acc0 + seq_v[ob + l, pl.ds(0, 16)]
                acc1 = acc1 + seq_v[ob + l, pl.ds(16, 16)]
            cf = cnt.astype(jnp.float32)
            nz = 50.0 - cf
            denom = jnp.maximum(cf, 1.0)
            p0 = jnp.where(cnt > 0, (acc0 - nz * r0a) / denom, 0.0)
            p1 = jnp.where(cnt > 0, (acc1 - nz * r0b) / denom, 0.0)
            orow = s * (F + 1) + F
            rows_v[orow, pl.ds(0, 16)] = p0
            rows_v[orow, pl.ds(16, 16)] = p1

        pltpu.async_copy(rows_v, out.at[pl.ds(s_base * (F + 1), ROWS)], osem)

    stage1(0, b0)

    def body(k, carry):
        c0 = 2 * k
        stage1(c0 + 1, b1)
        stage2(c0, b0)

        @pl.when(c0 + 2 < NCHUNK)
        def _():
            stage1(c0 + 2, b0)

        stage2(c0 + 1, b1)
        return carry

    lax.fori_loop(0, NCHUNK // 2, body, 0)

    last0 = (wid * SPW + (NCHUNK - 2) * S) * (F + 1)
    last1 = (wid * SPW + (NCHUNK - 1) * S) * (F + 1)
    pltpu.make_async_copy(b0[3], out.at[pl.ds(last0, ROWS)], b0[6]).wait()
    pltpu.make_async_copy(b1[3], out.at[pl.ds(last1, ROWS)], b1[6]).wait()


def kernel(sparse_ids, seq_ids, sparse_tables, seq_table):
    t832 = jnp.swapaxes(sparse_tables, 1, 2).reshape(F * D, V)
    seqT = jnp.swapaxes(seq_table, 0, 1)
    ftab, fseq = _sc_detile(t832, seqT)
    spT = jnp.swapaxes(sparse_ids.astype(jnp.int32), 0, 1)
    sq = seq_ids.astype(jnp.int32).reshape(B * L)
    out = _sc_embed(spT, sq, ftab.reshape(F * VP, D), fseq.reshape(VP, D))
    return out.reshape(B, F + 1, D)


# speedup vs baseline: 2.9630x; 1.0996x over previous
import functools

import jax
import jax.numpy as jnp
from jax import lax
from jax.experimental import pallas as pl
from jax.experimental.pallas import tpu as pltpu
from jax.experimental.pallas import tpu_sc as plsc

B = 16384
F = 26
L = 50
V = 100000
D = 32

NC = 2
NS = 16
NW = NC * NS
SPW = B // NW
S = 16
NCHUNK = SPW // S
ROWS = S * (F + 1)
SEQN = S * L
GSUB = 128

VP = 100352
WU = 512
VBN = 196
NUNIT = (F + 1) * VBN
UPT = 166

_mesh = plsc.VectorSubcoreMesh(core_axis_name="c", subcore_axis_name="s")

_k1_buf = [
    pltpu.VMEM((D, WU), jnp.float32),
    pltpu.VMEM((D, 32), jnp.float32),
    pltpu.VMEM((WU * D,), jnp.float32),
    pltpu.SemaphoreType.DMA,
    pltpu.SemaphoreType.DMA,
]


@functools.partial(
    pl.kernel,
    out_type=(jax.ShapeDtypeStruct((F * VP * D,), jnp.float32),
              jax.ShapeDtypeStruct((VP * D,), jnp.float32)),
    mesh=_mesh,
    compiler_params=pltpu.CompilerParams(
        needs_layout_passes=False, use_tc_tiling_on_sc=True),
    scratch_types=_k1_buf + _k1_buf,
)
def _sc_detile(t832, seqT, out_tab, out_seq, *scr):
    bufs = (scr[:5], scr[5:])
    wid = lax.axis_index("s") * NC + lax.axis_index("c")
    lanes32 = lax.iota(jnp.int32, 16) * D

    def decode(k):
        u = wid + k * NW
        return u, u // VBN, u % VBN

    def in_copy(f, vb, in_v, tail_v, isem, wait):
        def go(src, r0):
            @pl.when(vb < VBN - 1)
            def _():
                cp = pltpu.make_async_copy(
                    src.at[pl.ds(r0, D), pl.ds(vb * WU, WU)], in_v, isem)
                cp.wait() if wait else cp.start()

            @pl.when(vb == VBN - 1)
            def _():
                t0 = (VBN - 1) * WU
                cps = [
                    pltpu.make_async_copy(
                        src.at[pl.ds(r0, D), pl.ds(t0, 128)],
                        in_v.at[pl.ds(0, D), pl.ds(0, 128)], isem),
                    pltpu.make_async_copy(
                        src.at[pl.ds(r0, D), pl.ds(t0 + 128, V - t0 - 128)],
                        tail_v, isem),
                ]
                for cp in cps:
                    cp.wait() if wait else cp.start()

        @pl.when(f < F)
        def _():
            go(t832, f * D)

        @pl.when(f == F)
        def _():
            go(seqT, 0)

    def out_copy(f, vb, out_v, osem, wait):
        @pl.when(f < F)
        def _():
            cp = pltpu.make_async_copy(
                out_v, out_tab.at[pl.ds((f * VP + vb * WU) * D, WU * D)], osem)
            cp.wait() if wait else cp.start()

        @pl.when(f == F)
        def _():
            cp = pltpu.make_async_copy(
                out_v, out_seq.at[pl.ds(vb * WU * D, WU * D)], osem)
            cp.wait() if wait else cp.start()

    def transpose(vb, in_v, tail_v, out_v):
        lanes_ = lax.iota(jnp.int32, 16)

        def diag(src, sl):
            sl32 = sl * D
            for k in range(D):
                dvec = jnp.bitwise_and(lanes_ + k, D - 1)
                val = plsc.load_gather(src, [dvec, sl])
                plsc.store_scatter(out_v, [sl32 + dvec], val)

        @pl.when(vb < VBN - 1)
        def _():
            @plsc.parallel_loop(0, WU // 16, unroll=4)
            def _(seg):
                diag(in_v, lanes_ + seg * 16)

        @pl.when(vb == VBN - 1)
        def _():
            @plsc.parallel_loop(0, 8, unroll=2)
            def _(seg):
                diag(in_v, lanes_ + seg * 16)
            for seg in (8, 9):
                sl = lanes_ + seg * 16
                sl32 = sl * D
                for k in range(D):
                    dvec = jnp.bitwise_and(lanes_ + k, D - 1)
                    val = plsc.load_gather(tail_v, [dvec, sl - 128])
                    plsc.store_scatter(out_v, [sl32 + dvec], val)

    def stage1(k, bp):
        in_v, tail_v, out_v, isem, osem = bp
        u, f, vb = decode(k)

        @pl.when(u < NUNIT)
        def _():
            in_copy(f, vb, in_v, tail_v, isem, wait=False)

    def stage2(k, bp):
        in_v, tail_v, out_v, isem, osem = bp
        u, f, vb = decode(k)

        @pl.when(u < NUNIT)
        def _():
            in_copy(f, vb, in_v, tail_v, isem, wait=True)

            @pl.when(k >= 2)
            def _():
                _, f2, vb2 = decode(k - 2)
                out_copy(f2, vb2, out_v, osem, wait=True)

            transpose(vb, in_v, tail_v, out_v)
            out_copy(f, vb, out_v, osem, wait=False)

    stage1(0, bufs[0])

    def body(j, carry):
        k0 = 2 * j
        stage1(k0 + 1, bufs[1])
        stage2(k0, bufs[0])
        stage1(k0 + 2, bufs[0])
        stage2(k0 + 1, bufs[1])
        return carry

    lax.fori_loop(0, UPT // 2, body, 0)

    for kk in (UPT - 2, UPT - 1):
        u, f, vb = decode(kk)
        bp = bufs[kk % 2]

        @pl.when(u < NUNIT)
        def _():
            out_copy(f, vb, bp[2], bp[4], wait=True)


_k2_buf = [
    pltpu.VMEM((F, S), jnp.int32),
    pltpu.VMEM((ROWS,), jnp.int32),
    pltpu.VMEM((SEQN,), jnp.int32),
    pltpu.VMEM((ROWS, D), jnp.float32),
    pltpu.VMEM((SEQN, D), jnp.float32),
    pltpu.SemaphoreType.DMA,
    pltpu.SemaphoreType.DMA,
]


@functools.partial(
    pl.kernel,
    out_type=jax.ShapeDtypeStruct((B * (F + 1), D), jnp.float32),
    mesh=_mesh,
    compiler_params=pltpu.CompilerParams(
        needs_layout_passes=False, use_tc_tiling_on_sc=False),
    scratch_types=_k2_buf + _k2_buf + [
        pltpu.VMEM((1, D), jnp.float32),
    ],
)
def _sc_embed(spT, sq_ids, tables, seq_tab, out, *scr):
    b0, b1 = scr[:7], scr[7:14]
    row0_v = scr[14]
    wid = lax.axis_index("s") * NC + lax.axis_index("c")
    pltpu.sync_copy(seq_tab.at[pl.ds(0, 1)], row0_v)
    lanes = lax.iota(jnp.int32, 16)
    lanes27 = lanes * (F + 1)

    def stage1(c, bufs):
        ids2_v, idx_v, sqid_v, rows_v, seq_v, gsem, osem = bufs
        s_base = wid * SPW + c * S
        pltpu.sync_copy(spT.at[pl.ds(0, F), pl.ds(s_base, S)], ids2_v)
        pltpu.sync_copy(sq_ids.at[pl.ds(s_base * L, SEQN)], sqid_v)

        for f in range(F):
            idv = ids2_v[f, pl.ds(0, 16)]
            plsc.store_scatter(idx_v, [lanes27 + f], idv + f * VP)
        plsc.store_scatter(idx_v, [lanes27 + F], jnp.zeros((16,), jnp.int32))

        @pl.when(c >= 2)
        def _():
            old = (wid * SPW + (c - 2) * S) * (F + 1)
            pltpu.make_async_copy(rows_v, out.at[pl.ds(old, ROWS)], osem).wait()

        for off in range(0, ROWS, GSUB):
            n = min(GSUB, ROWS - off)
            pltpu.async_copy(
                tables.at[idx_v.at[pl.ds(off, n)]], rows_v.at[pl.ds(off, n)], gsem)
        for off in range(0, SEQN, GSUB):
            n = min(GSUB, SEQN - off)
            pltpu.async_copy(
                seq_tab.at[sqid_v.at[pl.ds(off, n)]], seq_v.at[pl.ds(off, n)], gsem)

    def stage2(c, bufs):
        ids2_v, idx_v, sqid_v, rows_v, seq_v, gsem, osem = bufs
        s_base = wid * SPW + c * S
        for off in range(0, ROWS, GSUB):
            n = min(GSUB, ROWS - off)
            pltpu.make_async_copy(
                tables.at[idx_v.at[pl.ds(off, n)]], rows_v.at[pl.ds(off, n)], gsem).wait()
        for off in range(0, SEQN, GSUB):
            n = min(GSUB, SEQN - off)
            pltpu.make_async_copy(
                seq_tab.at[sqid_v.at[pl.ds(off, n)]], seq_v.at[pl.ds(off, n)], gsem).wait()

        r0a = row0_v[0, pl.ds(0, 16)]
        r0b = row0_v[0, pl.ds(16, 16)]

        @plsc.parallel_loop(0, S)
        def sample_body(s):
            ob = s * L
            i0 = sqid_v[pl.ds(ob, 16)]
            i1 = sqid_v[pl.ds(ob + 16, 16)]
            i2 = sqid_v[pl.ds(ob + 32, 16)]
            i3 = sqid_v[pl.ds(ob + 34, 16)]
            cnt = (plsc.all_reduce_population_count(i0 != 0)
                   + plsc.all_reduce_population_count(i1 != 0)
                   + plsc.all_reduce_population_count(i2 != 0)
                   + plsc.all_reduce_population_count(
                       jnp.logical_and(i3 != 0, lanes >= 14)))
            acc0 = jnp.zeros((16,), jnp.float32)
            acc1 = jnp.zeros((16,), jnp.float32)
            for l in range(L):
                acc0 = acc0 + seq_v[ob + l, pl.ds(0, 16)]
                acc1 = acc1 + seq_v[ob + l, pl.ds(16, 16)]
            cf = cnt.astype(jnp.float32)
            nz = 50.0 - cf
            denom = jnp.maximum(cf, 1.0)
            p0 = jnp.where(cnt > 0, (acc0 - nz * r0a) / denom, 0.0)
            p1 = jnp.where(cnt > 0, (acc1 - nz * r0b) / denom, 0.0)
            orow = s * (F + 1) + F
            rows_v[orow, pl.ds(0, 16)] = p0
            rows_v[orow, pl.ds(16, 16)] = p1

        pltpu.async_copy(rows_v, out.at[pl.ds(s_base * (F + 1), ROWS)], osem)

    stage1(0, b0)

    def body(k, carry):
        c0 = 2 * k
        stage1(c0 + 1, b1)
        stage2(c0, b0)

        @pl.when(c0 + 2 < NCHUNK)
        def _():
            stage1(c0 + 2, b0)

        stage2(c0 + 1, b1)
        return carry

    lax.fori_loop(0, NCHUNK // 2, body, 0)

    last0 = (wid * SPW + (NCHUNK - 2) * S) * (F + 1)
    last1 = (wid * SPW + (NCHUNK - 1) * S) * (F + 1)
    pltpu.make_async_copy(b0[3], out.at[pl.ds(last0, ROWS)], b0[6]).wait()
    pltpu.make_async_copy(b1[3], out.at[pl.ds(last1, ROWS)], b1[6]).wait()


SB = 64


@functools.partial(
    pl.kernel,
    out_type=jax.ShapeDtypeStruct((F + 1, D, B), jnp.float32),
    mesh=_mesh,
    compiler_params=pltpu.CompilerParams(
        needs_layout_passes=False, use_tc_tiling_on_sc=False),
    scratch_types=[
        pltpu.VMEM(((F + 1) * SB, D), jnp.float32),
        pltpu.VMEM((F + 1, D, SB), jnp.float32),
    ],
)
def _sc_outx(mid, outx, in_v, out_v):
    wid = lax.axis_index("s") * NC + lax.axis_index("c")
    lanes_ = lax.iota(jnp.int32, 16)

    def sub(sc, carry):
        b0 = wid * SPW + sc * SB
        pltpu.sync_copy(mid.at[pl.ds(b0 * (F + 1), SB * (F + 1))], in_v)

        @plsc.parallel_loop(0, (F + 1) * (SB // 16))
        def _(j):
            fp1 = j // (SB // 16)
            sb16l = (j % (SB // 16)) * 16 + lanes_
            rowv = sb16l * (F + 1) + fp1
            fp1v = jnp.zeros((16,), jnp.int32) + fp1
            for k in range(D):
                dvec = jnp.bitwise_and(lanes_ + k, D - 1)
                val = plsc.load_gather(in_v, [rowv, dvec])
                plsc.store_scatter(out_v, [fp1v, dvec, sb16l], val)

        pltpu.sync_copy(out_v, outx.at[pl.ds(0, F + 1), pl.ds(0, D), pl.ds(b0, SB)])
        return carry

    lax.fori_loop(0, SPW // SB, sub, 0)


def kernel(sparse_ids, seq_ids, sparse_tables, seq_table):
    t832 = jnp.swapaxes(sparse_tables, 1, 2).reshape(F * D, V)
    seqT = jnp.swapaxes(seq_table, 0, 1)
    ftab, fseq = _sc_detile(t832, seqT)
    spT = jnp.swapaxes(sparse_ids.astype(jnp.int32), 0, 1)
    sq = seq_ids.astype(jnp.int32).reshape(B * L)
    mid = _sc_embed(spT, sq, ftab.reshape(F * VP, D), fseq.reshape(VP, D))
    return _sc_outx(mid).transpose(2, 0, 1)
